# baseline (speedup 1.0000x reference)
import jax
import jax.numpy as jnp
from jax.experimental import pallas as pl
from jax.experimental.pallas import tpu as pltpu


def _mlp_gate_block(x_ref, pe_ref, w1_ref, w2_ref, b1_ref, b2_ref,
                    coff_ref, t_ref, o_ref, pe_s, w2_s):
    d = w2_ref.shape[0]

    @pl.when(pl.program_id(0) == 0)
    def _prep():
        trow = t_ref[0, 0] + 1
        pe_s[...] = pe_ref[pl.ds(trow, 1), :] * coff_ref[0, 0]
        w2_s[...] = jnp.broadcast_to(w2_ref[:, 1:2], (d, d))

    x = x_ref[...] + pe_s[...]
    h = jnp.dot(x, w1_ref[...], preferred_element_type=jnp.float32)
    h = jnp.maximum(h + b1_ref[...], 0.0)
    m = jnp.dot(h, w2_s[...], preferred_element_type=jnp.float32)
    m = jnp.maximum(m + b2_ref[0, 1], 0.0)
    o_ref[...] = x * jax.nn.sigmoid(m)


def kernel(graph, input, W1, b1, W2, b2, pe_coff, pe, t):
    n, d = input.shape
    block = 5000
    assert n % block == 0
    grid = (n // block,)

    vmem = pl.BlockSpec(memory_space=pltpu.VMEM)
    smem = pl.BlockSpec(memory_space=pltpu.SMEM)
    return pl.pallas_call(
        _mlp_gate_block,
        grid=grid,
        in_specs=[
            pl.BlockSpec((block, d), lambda i: (i, 0)),
            vmem, vmem, vmem, vmem,
            smem, smem, smem,
        ],
        out_specs=pl.BlockSpec((block, d), lambda i: (i, 0)),
        out_shape=jax.ShapeDtypeStruct((n, d), jnp.float32),
        scratch_shapes=[
            pltpu.VMEM((1, d), jnp.float32),
            pltpu.VMEM((d, d), jnp.float32),
        ],
        compiler_params=pltpu.CompilerParams(
            dimension_semantics=("arbitrary",),
        ),
    )(
        input, pe, W1, W2,
        b1.reshape(1, d),
        b2.reshape(1, 2),
        pe_coff.reshape(1, 1),
        jnp.asarray(t, jnp.int32).reshape(1, 1),
    )


# speedup vs baseline: 1.1426x; 1.1426x over previous
import jax
import jax.numpy as jnp
from jax.experimental import pallas as pl
from jax.experimental.pallas import tpu as pltpu


def _mlp_gate_block(x_ref, pe_ref, w1_ref, w2_ref, b1_ref, b2_ref,
                    coff_ref, t_ref, o_ref, pe_s, w2_s):
    d = w2_ref.shape[0]

    @pl.when(pl.program_id(0) == 0)
    def _prep():
        trow = t_ref[0, 0] + 1
        pe_s[...] = pe_ref[pl.ds(trow, 1), :] * coff_ref[0, 0]
        w2_s[...] = jnp.broadcast_to(w2_ref[:, 1:2], (d, d))

    x = x_ref[...] + pe_s[...]
    h = jnp.dot(x, w1_ref[...], preferred_element_type=jnp.float32)
    h = jnp.maximum(h + b1_ref[...], 0.0)
    m = jnp.dot(h, w2_s[...], preferred_element_type=jnp.float32)
    m = jnp.maximum(m + b2_ref[0, 1], 0.0)
    o_ref[...] = x * jax.nn.sigmoid(m)


def kernel(graph, input, W1, b1, W2, b2, pe_coff, pe, t):
    n, d = input.shape
    block = 25000
    assert n % block == 0
    grid = (n // block,)

    vmem = pl.BlockSpec(memory_space=pltpu.VMEM)
    smem = pl.BlockSpec(memory_space=pltpu.SMEM)
    return pl.pallas_call(
        _mlp_gate_block,
        grid=grid,
        in_specs=[
            pl.BlockSpec((block, d), lambda i: (i, 0)),
            vmem, vmem, vmem, vmem,
            smem, smem, smem,
        ],
        out_specs=pl.BlockSpec((block, d), lambda i: (i, 0)),
        out_shape=jax.ShapeDtypeStruct((n, d), jnp.float32),
        scratch_shapes=[
            pltpu.VMEM((1, d), jnp.float32),
            pltpu.VMEM((d, d), jnp.float32),
        ],
        compiler_params=pltpu.CompilerParams(
            dimension_semantics=("arbitrary",),
            vmem_limit_bytes=67108864,
        ),
    )(
        input, pe, W1, W2,
        b1.reshape(1, d),
        b2.reshape(1, 2),
        pe_coff.reshape(1, 1),
        jnp.asarray(t, jnp.int32).reshape(1, 1),
    )


# speedup vs baseline: 1.2448x; 1.0894x over previous
import jax
import jax.numpy as jnp
from jax.experimental import pallas as pl
from jax.experimental.pallas import tpu as pltpu


def _mlp_gate_block(x_ref, pe_ref, w1_ref, w2_ref, b1_ref, b2_ref,
                    coff_ref, t_ref, o_ref, pe_s, w2_s):
    d = w2_ref.shape[0]

    @pl.when(pl.program_id(0) == 0)
    def _prep():
        trow = t_ref[0, 0] + 1
        pe_s[...] = pe_ref[pl.ds(trow, 1), :] * coff_ref[0, 0]
        w2_s[...] = jnp.broadcast_to(w2_ref[:, 1:2], (d, d))

    x = x_ref[...] + pe_s[...]
    h = jnp.dot(x, w1_ref[...], preferred_element_type=jnp.float32)
    h = jnp.maximum(h + b1_ref[...], 0.0)
    m = jnp.dot(h, w2_s[...], preferred_element_type=jnp.float32)
    m = jnp.maximum(m + b2_ref[0, 1], 0.0)
    o_ref[...] = x / (1.0 + jnp.exp(-m))


def kernel(graph, input, W1, b1, W2, b2, pe_coff, pe, t):
    n, d = input.shape
    block = 20000
    assert n % block == 0
    grid = (n // block,)

    vmem = pl.BlockSpec(memory_space=pltpu.VMEM)
    smem = pl.BlockSpec(memory_space=pltpu.SMEM)
    return pl.pallas_call(
        _mlp_gate_block,
        grid=grid,
        in_specs=[
            pl.BlockSpec((block, d), lambda i: (i, 0)),
            vmem, vmem, vmem, vmem,
            smem, smem, smem,
        ],
        out_specs=pl.BlockSpec((block, d), lambda i: (i, 0)),
        out_shape=jax.ShapeDtypeStruct((n, d), jnp.float32),
        scratch_shapes=[
            pltpu.VMEM((1, d), jnp.float32),
            pltpu.VMEM((d, d), jnp.float32),
        ],
        compiler_params=pltpu.CompilerParams(
            dimension_semantics=("arbitrary",),
        ),
    )(
        input, pe, W1, W2,
        b1.reshape(1, d),
        b2.reshape(1, 2),
        pe_coff.reshape(1, 1),
        jnp.asarray(t, jnp.int32).reshape(1, 1),
    )


# speedup vs baseline: 1.2909x; 1.0370x over previous
import jax
import jax.numpy as jnp
from jax.experimental import pallas as pl
from jax.experimental.pallas import tpu as pltpu

_BLOCK = 20000
_SUB = 4000
_NSUB = _BLOCK // _SUB


def _body(x_ref, pe_ref, w1_ref, w2_ref, b1_ref, b2_ref, coff_ref, t_ref,
          o_hbm, pe_s, w2_s, obuf, osems):
    d = w2_ref.shape[0]
    i = pl.program_id(0)

    @pl.when(i == 0)
    def _prep():
        trow = t_ref[0, 0] + 1
        pe_s[...] = pe_ref[pl.ds(trow, 1), :] * coff_ref[0, 0]
        w2_s[...] = jnp.broadcast_to(w2_ref[:, 1:2], (d, d))

    for s in range(_NSUB):
        def out_copy(row0, slot):
            return pltpu.make_async_copy(
                obuf.at[slot], o_hbm.at[pl.ds(row0, _SUB), :], osems.at[slot])

        @pl.when(i > 0)
        def _retire():
            out_copy((i - 1) * _BLOCK + s * _SUB, s).wait()

        x = x_ref[pl.ds(s * _SUB, _SUB), :] + pe_s[...]
        h = jnp.dot(x, w1_ref[...], preferred_element_type=jnp.float32)
        h = jnp.maximum(h + b1_ref[...], 0.0)
        m = jnp.dot(h, w2_s[...], preferred_element_type=jnp.float32)
        m = jnp.maximum(m + b2_ref[0, 1], 0.0)
        obuf[s] = x * jax.nn.sigmoid(m)
        out_copy(i * _BLOCK + s * _SUB, s).start()

    @pl.when(i == pl.num_programs(0) - 1)
    def _drain():
        for s in range(_NSUB):
            pltpu.make_async_copy(
                obuf.at[s], o_hbm.at[pl.ds(s * _SUB, _SUB), :],
                osems.at[s]).wait()


def kernel(graph, input, W1, b1, W2, b2, pe_coff, pe, t):
    n, d = input.shape
    assert n % _BLOCK == 0
    grid = (n // _BLOCK,)

    vmem = pl.BlockSpec(memory_space=pltpu.VMEM)
    smem = pl.BlockSpec(memory_space=pltpu.SMEM)
    return pl.pallas_call(
        _body,
        grid=grid,
        in_specs=[
            pl.BlockSpec((_BLOCK, d), lambda i: (i, 0)),
            vmem, vmem, vmem, vmem,
            smem, smem, smem,
        ],
        out_specs=pl.BlockSpec(memory_space=pl.ANY),
        out_shape=jax.ShapeDtypeStruct((n, d), jnp.float32),
        scratch_shapes=[
            pltpu.VMEM((1, d), jnp.float32),
            pltpu.VMEM((d, d), jnp.float32),
            pltpu.VMEM((_NSUB, _SUB, d), jnp.float32),
            pltpu.SemaphoreType.DMA((_NSUB,)),
        ],
        compiler_params=pltpu.CompilerParams(
            dimension_semantics=("arbitrary",),
        ),
    )(
        input, pe, W1, W2,
        b1.reshape(1, d),
        b2.reshape(1, 2),
        pe_coff.reshape(1, 1),
        jnp.asarray(t, jnp.int32).reshape(1, 1),
    )
